# baseline (speedup 1.0000x reference)
import functools
import math

import jax
import jax.numpy as jnp
from jax import lax
from jax.experimental import pallas as pl
from jax.experimental.pallas import tpu as pltpu
from jax.experimental.pallas import tpu_sc as plsc

N = 10000
E = 160000
H = 8
DH = 128
M = 30

NP = 11264
EP = 160256
SENT = 10760
PASSES = ((0, 2048), (2048, 2048), (4096, 2048), (6144, 2048), (8192, 2048))
AGG_ROWS = 10240
BN = 1000
NB = N // BN
DN = 1.0 / (128.0 ** 0.25)
RATIO = 1.0 / math.sqrt(M)
NEG_SLOPE = 0.01
EPS = 1e-5

_F32 = jnp.float32
_I32 = jnp.int32


def _sc_degrees(cols_p, rows_p):
    SL = EP // 32
    RPW = NP // 16
    mesh = plsc.VectorSubcoreMesh(core_axis_name="c", subcore_axis_name="s")

    @functools.partial(
        pl.kernel, mesh=mesh,
        out_type=jax.ShapeDtypeStruct((4 * NP,), _F32),
        scratch_types=[
            pltpu.VMEM((SL,), _I32),
            pltpu.VMEM((NP,), _F32),
            pltpu.VMEM((NP,), _F32),
            pltpu.VMEM((RPW,), _F32),
            pltpu.VMEM((RPW,), _F32),
            pltpu.VMEM_SHARED((16 * 2 * NP,), _F32),
        ],
    )
    def k(cols_hbm, rows_hbm, out_hbm, ev, tin, tout, part, racc, sh):
        c = lax.axis_index("c")
        s = lax.axis_index("s")
        w = c * 16 + s
        iota16 = lax.iota(_I32, 16)
        zero16 = jnp.zeros((16,), _F32)

        def zlarge(i, carry):
            tin[pl.ds(i * 16, 16)] = zero16
            tout[pl.ds(i * 16, 16)] = zero16
            return carry

        lax.fori_loop(0, NP // 16, zlarge, 0)

        for arr, tbl in ((cols_hbm, tin), (rows_hbm, tout)):
            pltpu.sync_copy(arr.at[pl.ds(w * SL, SL)], ev)

            def hbody(i, carry, _tbl=tbl):
                e16 = ev[pl.ds(i * 16, 16)]
                for j in range(16):
                    li = e16[j]
                    w16 = (li >> 4) << 4
                    oh = jnp.where(iota16 == (li - w16), 1.0, 0.0)
                    plsc.addupdate(_tbl.at[pl.ds(w16, 16)], oh)
                return carry

            lax.fori_loop(0, SL // 16, hbody, 0)

        pltpu.sync_copy(tin, sh.at[pl.ds(s * 2 * NP, NP)])
        pltpu.sync_copy(tout, sh.at[pl.ds(s * 2 * NP + NP, NP)])
        plsc.subcore_barrier()

        for t in range(2):
            def zr(i, carry):
                racc[pl.ds(i * 16, 16)] = zero16
                return carry

            lax.fori_loop(0, RPW // 16, zr, 0)
            for ww in range(16):
                pltpu.sync_copy(
                    sh.at[pl.ds(ww * 2 * NP + t * NP + s * RPW, RPW)], part)

                def radd(i, carry):
                    racc[pl.ds(i * 16, 16)] += part[pl.ds(i * 16, 16)]
                    return carry

                lax.fori_loop(0, RPW // 16, radd, 0)
            pltpu.sync_copy(
                racc,
                out_hbm.at[pl.ds(c * 2 * NP + t * NP + s * RPW, RPW)])

    return k(cols_p, rows_p)


def _sc_spmm(vp2, cols_p, rows_p):
    K = 32
    RPT = 128
    TROWS = 129
    SEG = 8192
    NSEG = 19
    TAILV = (EP - NSEG * SEG) // 16
    mesh = plsc.VectorSubcoreMesh(core_axis_name="c", subcore_axis_name="s")

    @functools.partial(
        pl.kernel, mesh=mesh,
        out_type=jax.ShapeDtypeStruct((2 * AGG_ROWS * 512,), _F32),
        scratch_types=[
            pltpu.VMEM((SEG,), _I32),
            pltpu.VMEM((SEG,), _I32),
            pltpu.VMEM((SEG + 2 * K,), _I32),
            pltpu.VMEM((SEG + 2 * K,), _I32),
            pltpu.VMEM((K,), _I32),
            pltpu.VMEM((K, 512), _F32),
            pltpu.VMEM((TROWS * 512,), _F32),
            pltpu.SemaphoreType.DMA,
        ],
    )
    def k(vp_hbm, cols_hbm, rows_hbm, out_hbm,
          segc, segr, fidx, frow, gidx, gbuf, tbl, sem):
        c = lax.axis_index("c")
        s = lax.axis_index("s")
        iota16 = lax.iota(_I32, 16)
        zero16 = jnp.zeros((16,), _F32)
        zi16 = jnp.zeros((16,), _I32)
        dump16 = jnp.full((16,), RPT, _I32)
        goff = c * N

        def process_segment(eoff, nv, lo):
            ne = nv * 16
            pltpu.sync_copy(cols_hbm.at[pl.ds(eoff, ne)],
                            segc.at[pl.ds(0, ne)])
            pltpu.sync_copy(rows_hbm.at[pl.ds(eoff, ne)],
                            segr.at[pl.ds(0, ne)])

            def fbody(i, off):
                c16 = segc[pl.ds(i * 16, 16)]
                msk = (c16 >= lo) & (c16 < lo + RPT)
                mi = jnp.where(msk, 1, 0).astype(_I32)
                incl = mi
                for d in (1, 2, 4, 8):
                    y = incl[jnp.maximum(iota16 - d, 0)]
                    incl = incl + jnp.where(iota16 >= d, y, 0)
                cnt = incl[15]

                @pl.when(cnt > 0)
                def _():
                    r16 = segr[pl.ds(i * 16, 16)]
                    excl = incl - mi
                    cl = c16 - lo
                    for j in range(16):
                        d_j = off + excl[j]
                        fidx[pl.ds(d_j, 16)] = jnp.full((16,), cl[j], _I32)
                        frow[pl.ds(d_j, 16)] = jnp.full((16,), r16[j], _I32)

                return off + cnt

            nf = lax.fori_loop(0, nv, fbody, jnp.int32(0))

            for jp in range(K // 16):
                fidx[pl.ds(nf + jp * 16, 16)] = dump16
                frow[pl.ds(nf + jp * 16, 16)] = zi16

            def gbody(g, carry):
                for jv in range(K // 16):
                    gidx[pl.ds(jv * 16, 16)] = (
                        frow[pl.ds(g * K + jv * 16, 16)] + goff)
                pltpu.async_copy(vp_hbm.at[gidx], gbuf, sem).wait()
                for jv in range(K // 16):
                    lv = fidx[pl.ds(g * K + jv * 16, 16)]
                    for j in range(16):
                        rowoff = lv[j] * 512

                        def rmw(q, cc, _j=jv * 16 + j, _ro=rowoff):
                            plsc.addupdate(tbl.at[pl.ds(_ro + q * 16, 16)],
                                           gbuf[_j, pl.ds(q * 16, 16)])
                            return cc

                        lax.fori_loop(0, 32, rmw, 0)
                return carry

            lax.fori_loop(0, (nf + (K - 1)) // K, gbody, 0)

        for base, creal in PASSES:
            lo = base + s * RPT

            def ztbl(i, carry):
                tbl[pl.ds(i * 16, 16)] = zero16
                return carry

            lax.fori_loop(0, TROWS * 32, ztbl, 0)

            def seg_loop(g, carry, _lo=lo):
                process_segment(g * SEG, SEG // 16, _lo)
                return carry

            lax.fori_loop(0, NSEG, seg_loop, 0)
            process_segment(NSEG * SEG, TAILV, lo)

            pltpu.sync_copy(
                tbl.at[pl.ds(0, RPT * 512)],
                out_hbm.at[pl.ds((c * AGG_ROWS + base) * 512 + s * RPT * 512,
                                 RPT * 512)])

    return k(vp2, cols_p, rows_p)


def _norm_apply(x, stats_ref, nw_ref, nb_ref, nms_ref):
    sy = stats_ref[0:1, :]
    sy2 = stats_ref[1:2, :]
    mean = sy * (1.0 / N)
    ey2 = sy2 * (1.0 / N)
    m2 = mean * nms_ref[0:1, :]
    var = ey2 - 2.0 * m2 * mean + m2 * m2
    return nw_ref[0:1, :] * (x - m2) * lax.rsqrt(var + EPS) + nb_ref[0:1, :]


def _lane_lt(n):
    return lax.broadcasted_iota(_I32, (BN, 128), 1) < n


def _k1_body(first, *refs):
    if first:
        (x_ref, w3_ref, b3_ref, pjt_ref, deg_ref,
         qp_ref, sk_ref, kmax_ref, v_ref, vp_ref, ab_ref) = refs
        x = x_ref[...]
    else:
        (x_ref, w3_ref, b3_ref, pjt_ref, deg_ref,
         stats_ref, nw_ref, nb_ref, nms_ref,
         qp_ref, sk_ref, kmax_ref, v_ref, vp_ref, ab_ref) = refs
        x = _norm_apply(x_ref[...], stats_ref, nw_ref, nb_ref, nms_ref)

    qkv = jnp.dot(x, w3_ref[...], preferred_element_type=_F32) + b3_ref[0:1, :]
    deg = deg_ref[...]
    din = deg[:, 0:1] + deg[:, 1:2]
    dout = deg[:, 2:3] + deg[:, 3:4]
    a_col = lax.rsqrt(jnp.maximum(din, 1.0))
    b_row = lax.rsqrt(jnp.maximum(dout, 1.0))
    v = qkv[:, 2048:3072]
    v_ref[...] = v
    vpn = v * b_row
    vp_ref[0] = vpn[:, :512]
    vp_ref[1] = vpn[:, 512:]
    ab_ref[...] = jnp.concatenate(
        [a_col, b_row, jnp.zeros((BN, 126), _F32)], axis=1)

    pjt = pjt_ref[...]
    m30 = _lane_lt(M)
    neg_inf = jnp.float32(-jnp.inf)
    kmax = None
    for h in range(H):
        qh = qkv[:, h * 128:(h + 1) * 128] * DN
        kh = qkv[:, 1024 + h * 128:1024 + (h + 1) * 128] * DN
        ddq = jnp.dot(qh, pjt, preferred_element_type=_F32)
        ddk = jnp.dot(kh, pjt, preferred_element_type=_F32)
        diagq = 0.5 * jnp.sum(qh * qh, axis=1, keepdims=True)
        diagk = 0.5 * jnp.sum(kh * kh, axis=1, keepdims=True)
        ddq_m = jnp.where(m30, ddq, neg_inf)
        stabq = jnp.max(ddq_m, axis=1, keepdims=True)
        qp = RATIO * (jnp.exp(ddq - diagq - stabq) + 1e-6)
        qp_ref[h] = jnp.where(m30, qp, 0.0)
        sk_ref[h] = jnp.where(m30, ddk - diagk, 0.0)
        mh = jnp.max(jnp.where(m30, ddk, neg_inf))
        kmax = mh if kmax is None else jnp.maximum(kmax, mh)
    kmax_ref[...] = jnp.broadcast_to(kmax, (1, 8, 128))


def _tc_k1(x, w3, b3, pjt, deg, norm):
    first = norm is None
    outs = (
        jax.ShapeDtypeStruct((H, N, 128), _F32),
        jax.ShapeDtypeStruct((H, N, 128), _F32),
        jax.ShapeDtypeStruct((NB, 8, 128), _F32),
        jax.ShapeDtypeStruct((N, 1024), _F32),
        jax.ShapeDtypeStruct((2, N, 512), _F32),
        jax.ShapeDtypeStruct((N, 128), _F32),
    )
    in_specs = [
        pl.BlockSpec((BN, 128), lambda i: (i, 0)),
        pl.BlockSpec((128, 3072), lambda i: (0, 0)),
        pl.BlockSpec((1, 3072), lambda i: (0, 0)),
        pl.BlockSpec((128, 128), lambda i: (0, 0)),
        pl.BlockSpec((BN, 128), lambda i: (i, 0)),
    ]
    args = [x, w3, b3, pjt, deg]
    if not first:
        stats, nw, nb_, nms = norm
        in_specs += [
            pl.BlockSpec((8, 128), lambda i: (0, 0)),
            pl.BlockSpec((1, 128), lambda i: (0, 0)),
            pl.BlockSpec((1, 128), lambda i: (0, 0)),
            pl.BlockSpec((1, 128), lambda i: (0, 0)),
        ]
        args += [stats, nw, nb_, nms]
    out_specs = (
        pl.BlockSpec((H, BN, 128), lambda i: (0, i, 0)),
        pl.BlockSpec((H, BN, 128), lambda i: (0, i, 0)),
        pl.BlockSpec((1, 8, 128), lambda i: (i, 0, 0)),
        pl.BlockSpec((BN, 1024), lambda i: (i, 0)),
        pl.BlockSpec((2, BN, 512), lambda i: (0, i, 0)),
        pl.BlockSpec((BN, 128), lambda i: (i, 0)),
    )
    return pl.pallas_call(
        functools.partial(_k1_body, first),
        grid=(NB,),
        in_specs=in_specs,
        out_specs=out_specs,
        out_shape=outs,
    )(*args)


def _k2_body(refs):
    sk_ref, kmax_ref, v_ref, kvs_ref, ks_ref = refs
    i = pl.program_id(0)
    gmax = jnp.max(kmax_ref[...])
    m30 = _lane_lt(M)

    @pl.when(i == 0)
    def _():
        kvs_ref[...] = jnp.zeros((H, 128, 128), _F32)
        ks_ref[...] = jnp.zeros((H, 128), _F32)

    ks_rows = []
    for h in range(H):
        kp = RATIO * (jnp.exp(sk_ref[h] - gmax) + 1e-6)
        kp = jnp.where(m30, kp, 0.0)
        vh = v_ref[:, h * 128:(h + 1) * 128]
        kvs_ref[h] += lax.dot_general(
            kp, vh, (((0,), (0,)), ((), ())), preferred_element_type=_F32)
        ks_rows.append(jnp.sum(kp, axis=0, keepdims=True))
    ks_ref[...] += jnp.concatenate(ks_rows, axis=0)


def _tc_k2(sk, kmax, v):
    return pl.pallas_call(
        lambda *refs: _k2_body(refs),
        grid=(NB,),
        in_specs=[
            pl.BlockSpec((H, BN, 128), lambda i: (0, i, 0)),
            pl.BlockSpec((NB, 8, 128), lambda i: (0, 0, 0)),
            pl.BlockSpec((BN, 1024), lambda i: (i, 0)),
        ],
        out_specs=(
            pl.BlockSpec((H, 128, 128), lambda i: (0, 0, 0)),
            pl.BlockSpec((H, 128), lambda i: (0, 0)),
        ),
        out_shape=(
            jax.ShapeDtypeStruct((H, 128, 128), _F32),
            jax.ShapeDtypeStruct((H, 128), _F32),
        ),
    )(sk, kmax, v)


def _k3_body(leaky, refs):
    (qp_ref, kvs_ref, ks_ref, agg0_ref, agg1_ref, ab_ref, rb_ref, wo_ref,
     bo_ref, y_ref, stats_ref) = refs
    i = pl.program_id(0)
    rb = rb_ref[...]
    a_col = ab_ref[:, 0:1]
    acc = jnp.zeros((BN, 128), _F32)
    for h in range(H):
        qp = qp_ref[h]
        numer = jnp.dot(qp, kvs_ref[h], preferred_element_type=_F32)
        denom = jnp.sum(qp * ks_ref[h:h + 1, :], axis=1, keepdims=True)
        bh = 1.0 / (1.0 + jnp.exp(-rb[0, h]))
        half_ref = agg0_ref if h < 4 else agg1_ref
        agg_h = half_ref[0][:, (h % 4) * 128:(h % 4 + 1) * 128]
        z = numer / denom + (a_col * bh) * agg_h
        acc += jnp.dot(z, wo_ref[h * 128:(h + 1) * 128, :],
                       preferred_element_type=_F32)
    y = acc + bo_ref[0:1, :]
    if leaky:
        y = jnp.where(y > 0, y, NEG_SLOPE * y)
    y_ref[...] = y
    sy = jnp.sum(y, axis=0, keepdims=True)
    sy2 = jnp.sum(y * y, axis=0, keepdims=True)
    st = jnp.concatenate([sy, sy2, jnp.zeros((6, 128), _F32)], axis=0)

    @pl.when(i == 0)
    def _():
        stats_ref[...] = st

    @pl.when(i > 0)
    def _():
        stats_ref[...] += st


def _tc_k3(qp, kvs, ks, agg, ab, rb, wo, bo, leaky):
    return pl.pallas_call(
        functools.partial(lambda lk, *refs: _k3_body(lk, refs), leaky),
        grid=(NB,),
        in_specs=[
            pl.BlockSpec((H, BN, 128), lambda i: (0, i, 0)),
            pl.BlockSpec((H, 128, 128), lambda i: (0, 0, 0)),
            pl.BlockSpec((H, 128), lambda i: (0, 0)),
            pl.BlockSpec((1, BN, 512), lambda i: (0, i, 0)),
            pl.BlockSpec((1, BN, 512), lambda i: (1, i, 0)),
            pl.BlockSpec((BN, 128), lambda i: (i, 0)),
            pl.BlockSpec((1, 128), lambda i: (0, 0)),
            pl.BlockSpec((1024, 128), lambda i: (0, 0)),
            pl.BlockSpec((1, 128), lambda i: (0, 0)),
        ],
        out_specs=(
            pl.BlockSpec((BN, 128), lambda i: (i, 0)),
            pl.BlockSpec((8, 128), lambda i: (0, 0)),
        ),
        out_shape=(
            jax.ShapeDtypeStruct((N, 128), _F32),
            jax.ShapeDtypeStruct((8, 128), _F32),
        ),
    )(qp, kvs, ks, agg, agg, ab, rb, wo, bo)


def _k4_body(y_ref, stats_ref, nw_ref, nb_ref, nms_ref, out_ref):
    out_ref[...] = _norm_apply(y_ref[...], stats_ref, nw_ref, nb_ref, nms_ref)


def _tc_k4(y, stats, nw, nb_, nms):
    return pl.pallas_call(
        _k4_body,
        grid=(NB,),
        in_specs=[
            pl.BlockSpec((BN, 128), lambda i: (i, 0)),
            pl.BlockSpec((8, 128), lambda i: (0, 0)),
            pl.BlockSpec((1, 128), lambda i: (0, 0)),
            pl.BlockSpec((1, 128), lambda i: (0, 0)),
            pl.BlockSpec((1, 128), lambda i: (0, 0)),
        ],
        out_specs=pl.BlockSpec((BN, 128), lambda i: (i, 0)),
        out_shape=jax.ShapeDtypeStruct((N, 128), _F32),
    )(y, stats, nw, nb_, nms)


def kernel(patch_embs, edge_index, edge_attr, params):
    del edge_attr
    rows = edge_index[0].astype(_I32)
    cols = edge_index[1].astype(_I32)
    pad = jnp.full((EP - E,), SENT, _I32)
    cols_p = jnp.concatenate([cols, pad])
    rows_p = jnp.concatenate([rows, pad])

    deg = _sc_degrees(cols_p, rows_p)
    degT = jnp.pad(
        jnp.stack([deg[:N], deg[2 * NP:2 * NP + N],
                   deg[NP:NP + N], deg[3 * NP:3 * NP + N]], axis=1),
        ((0, 0), (0, 124)))

    norm_keys = ("norm1", "norm2", "norm3")
    conv_keys = ("conv1", "conv2", "conv3")
    x = patch_embs
    norm = None
    stats = None
    for li in range(3):
        p = params[conv_keys[li]]
        w3 = jnp.concatenate([p["Wq"], p["Wk"], p["Wv"]], axis=1)
        b3 = jnp.concatenate([p["bq"], p["bk"], p["bv"]])[None, :]
        pjt = jnp.pad(p["proj"].T, ((0, 0), (0, 128 - M)))
        rb = jnp.pad(p["rb"], (0, 120))[None, :]
        qp, sk, kmax, v, vp, ab = _tc_k1(x, w3, b3, pjt, degT, norm)
        agg = _sc_spmm(vp.reshape(2 * N, 512), cols_p, rows_p)
        agg = agg.reshape(2, AGG_ROWS, 512)
        kvs, ks = _tc_k2(sk, kmax, v)
        y, stats = _tc_k3(qp, kvs, ks, agg, ab, rb, p["Wo"],
                          p["bo"][None, :], leaky=(li < 2))
        np_ = params[norm_keys[li]]
        norm = (stats, np_["w"][None, :], np_["b"][None, :],
                np_["ms"][None, :])
        x = y
    return _tc_k4(x, norm[0], norm[1], norm[2], norm[3])


# speedup vs baseline: 1.0164x; 1.0164x over previous
import functools
import math

import jax
import jax.numpy as jnp
from jax import lax
from jax.experimental import pallas as pl
from jax.experimental.pallas import tpu as pltpu
from jax.experimental.pallas import tpu_sc as plsc

N = 10000
E = 160000
H = 8
DH = 128
M = 30

NP = 11264
EP = 160256
SENT = 10760
PASSES = ((0, 2048), (2048, 2048), (4096, 2048), (6144, 2048), (8192, 2048))
AGG_ROWS = 10240
BN = 1000
NB = N // BN
DN = 1.0 / (128.0 ** 0.25)
RATIO = 1.0 / math.sqrt(M)
NEG_SLOPE = 0.01
EPS = 1e-5

_F32 = jnp.float32
_I32 = jnp.int32


def _sc_degrees(cols_p, rows_p):
    SL = EP // 32
    RPW = NP // 16
    mesh = plsc.VectorSubcoreMesh(core_axis_name="c", subcore_axis_name="s")

    @functools.partial(
        pl.kernel, mesh=mesh,
        out_type=jax.ShapeDtypeStruct((4 * NP,), _F32),
        scratch_types=[
            pltpu.VMEM((SL,), _I32),
            pltpu.VMEM((NP,), _F32),
            pltpu.VMEM((NP,), _F32),
            pltpu.VMEM((RPW,), _F32),
            pltpu.VMEM((RPW,), _F32),
            pltpu.VMEM_SHARED((16 * 2 * NP,), _F32),
        ],
    )
    def k(cols_hbm, rows_hbm, out_hbm, ev, tin, tout, part, racc, sh):
        c = lax.axis_index("c")
        s = lax.axis_index("s")
        w = c * 16 + s
        iota16 = lax.iota(_I32, 16)
        zero16 = jnp.zeros((16,), _F32)

        def zlarge(i, carry):
            tin[pl.ds(i * 16, 16)] = zero16
            tout[pl.ds(i * 16, 16)] = zero16
            return carry

        lax.fori_loop(0, NP // 16, zlarge, 0)

        for arr, tbl in ((cols_hbm, tin), (rows_hbm, tout)):
            pltpu.sync_copy(arr.at[pl.ds(w * SL, SL)], ev)

            def hbody(i, carry, _tbl=tbl):
                e16 = ev[pl.ds(i * 16, 16)]
                for j in range(16):
                    li = e16[j]
                    w16 = (li >> 4) << 4
                    oh = jnp.where(iota16 == (li - w16), 1.0, 0.0)
                    plsc.addupdate(_tbl.at[pl.ds(w16, 16)], oh)
                return carry

            lax.fori_loop(0, SL // 16, hbody, 0)

        pltpu.sync_copy(tin, sh.at[pl.ds(s * 2 * NP, NP)])
        pltpu.sync_copy(tout, sh.at[pl.ds(s * 2 * NP + NP, NP)])
        plsc.subcore_barrier()

        for t in range(2):
            def zr(i, carry):
                racc[pl.ds(i * 16, 16)] = zero16
                return carry

            lax.fori_loop(0, RPW // 16, zr, 0)
            for ww in range(16):
                pltpu.sync_copy(
                    sh.at[pl.ds(ww * 2 * NP + t * NP + s * RPW, RPW)], part)

                def radd(i, carry):
                    racc[pl.ds(i * 16, 16)] += part[pl.ds(i * 16, 16)]
                    return carry

                lax.fori_loop(0, RPW // 16, radd, 0)
            pltpu.sync_copy(
                racc,
                out_hbm.at[pl.ds(c * 2 * NP + t * NP + s * RPW, RPW)])

    return k(cols_p, rows_p)


def _sc_spmm(vp2, cols_p, rows_p):
    K = 16
    RPT = 128
    TROWS = 129
    SEG = 8192
    NSEG = 19
    TAILV = (EP - NSEG * SEG) // 16
    mesh = plsc.VectorSubcoreMesh(core_axis_name="c", subcore_axis_name="s")

    @functools.partial(
        pl.kernel, mesh=mesh,
        out_type=jax.ShapeDtypeStruct((2 * AGG_ROWS * 512,), _F32),
        scratch_types=[
            pltpu.VMEM((SEG,), _I32),
            pltpu.VMEM((SEG,), _I32),
            pltpu.VMEM((SEG + 2 * K,), _I32),
            pltpu.VMEM((SEG + 2 * K,), _I32),
            pltpu.VMEM((K,), _I32),
            pltpu.VMEM((K, 512), _F32),
            pltpu.VMEM((TROWS * 512,), _F32),
            pltpu.SemaphoreType.DMA,
        ],
    )
    def k(vp_hbm, cols_hbm, rows_hbm, out_hbm,
          segc, segr, fidx, frow, gidx, gbuf, tbl, sem):
        c = lax.axis_index("c")
        s = lax.axis_index("s")
        iota16 = lax.iota(_I32, 16)
        zero16 = jnp.zeros((16,), _F32)
        zi16 = jnp.zeros((16,), _I32)
        dump16 = jnp.full((16,), RPT, _I32)
        goff = c * N

        def process_segment(eoff, nv, lo):
            ne = nv * 16
            pltpu.sync_copy(cols_hbm.at[pl.ds(eoff, ne)],
                            segc.at[pl.ds(0, ne)])
            pltpu.sync_copy(rows_hbm.at[pl.ds(eoff, ne)],
                            segr.at[pl.ds(0, ne)])

            def fbody(i, off):
                c16 = segc[pl.ds(i * 16, 16)]
                msk = (c16 >= lo) & (c16 < lo + RPT)
                mi = jnp.where(msk, 1, 0).astype(_I32)
                incl = mi
                for d in (1, 2, 4, 8):
                    y = incl[jnp.maximum(iota16 - d, 0)]
                    incl = incl + jnp.where(iota16 >= d, y, 0)
                cnt = incl[15]

                @pl.when(cnt > 0)
                def _():
                    r16 = segr[pl.ds(i * 16, 16)]
                    excl = incl - mi
                    cl = c16 - lo
                    for j in range(16):
                        d_j = off + excl[j]
                        fidx[pl.ds(d_j, 16)] = jnp.full((16,), cl[j], _I32)
                        frow[pl.ds(d_j, 16)] = jnp.full((16,), r16[j], _I32)

                return off + cnt

            nf = lax.fori_loop(0, nv, fbody, jnp.int32(0))

            for jp in range(K // 16):
                fidx[pl.ds(nf + jp * 16, 16)] = dump16
                frow[pl.ds(nf + jp * 16, 16)] = zi16

            def gbody(g, carry):
                for jv in range(K // 16):
                    gidx[pl.ds(jv * 16, 16)] = (
                        frow[pl.ds(g * K + jv * 16, 16)] + goff)
                pltpu.async_copy(vp_hbm.at[gidx], gbuf, sem).wait()
                for jv in range(K // 16):
                    lv = fidx[pl.ds(g * K + jv * 16, 16)]
                    for j in range(16):
                        rowoff = lv[j] * 512

                        def rmw(q, cc, _j=jv * 16 + j, _ro=rowoff):
                            plsc.addupdate(tbl.at[pl.ds(_ro + q * 16, 16)],
                                           gbuf[_j, pl.ds(q * 16, 16)])
                            return cc

                        lax.fori_loop(0, 32, rmw, 0)
                return carry

            lax.fori_loop(0, (nf + (K - 1)) // K, gbody, 0)

        for base, creal in PASSES:
            lo = base + s * RPT

            def ztbl(i, carry):
                tbl[pl.ds(i * 16, 16)] = zero16
                return carry

            lax.fori_loop(0, TROWS * 32, ztbl, 0)

            def seg_loop(g, carry, _lo=lo):
                process_segment(g * SEG, SEG // 16, _lo)
                return carry

            lax.fori_loop(0, NSEG, seg_loop, 0)
            process_segment(NSEG * SEG, TAILV, lo)

            pltpu.sync_copy(
                tbl.at[pl.ds(0, RPT * 512)],
                out_hbm.at[pl.ds((c * AGG_ROWS + base) * 512 + s * RPT * 512,
                                 RPT * 512)])

    return k(vp2, cols_p, rows_p)


def _norm_apply(x, stats_ref, nw_ref, nb_ref, nms_ref):
    sy = stats_ref[0:1, :]
    sy2 = stats_ref[1:2, :]
    mean = sy * (1.0 / N)
    ey2 = sy2 * (1.0 / N)
    m2 = mean * nms_ref[0:1, :]
    var = ey2 - 2.0 * m2 * mean + m2 * m2
    return nw_ref[0:1, :] * (x - m2) * lax.rsqrt(var + EPS) + nb_ref[0:1, :]


def _lane_lt(n):
    return lax.broadcasted_iota(_I32, (BN, 128), 1) < n


def _k1_body(first, *refs):
    if first:
        (x_ref, w3_ref, b3_ref, pjt_ref, deg_ref,
         qp_ref, sk_ref, kmax_ref, v_ref, vp_ref, ab_ref) = refs
        x = x_ref[...]
    else:
        (x_ref, w3_ref, b3_ref, pjt_ref, deg_ref,
         stats_ref, nw_ref, nb_ref, nms_ref,
         qp_ref, sk_ref, kmax_ref, v_ref, vp_ref, ab_ref) = refs
        x = _norm_apply(x_ref[...], stats_ref, nw_ref, nb_ref, nms_ref)

    qkv = jnp.dot(x, w3_ref[...], preferred_element_type=_F32) + b3_ref[0:1, :]
    deg = deg_ref[...]
    din = deg[:, 0:1] + deg[:, 1:2]
    dout = deg[:, 2:3] + deg[:, 3:4]
    a_col = lax.rsqrt(jnp.maximum(din, 1.0))
    b_row = lax.rsqrt(jnp.maximum(dout, 1.0))
    v = qkv[:, 2048:3072]
    v_ref[...] = v
    vpn = v * b_row
    vp_ref[0] = vpn[:, :512]
    vp_ref[1] = vpn[:, 512:]
    ab_ref[...] = jnp.concatenate(
        [a_col, b_row, jnp.zeros((BN, 126), _F32)], axis=1)

    pjt = pjt_ref[...]
    m30 = _lane_lt(M)
    neg_inf = jnp.float32(-jnp.inf)
    kmax = None
    for h in range(H):
        qh = qkv[:, h * 128:(h + 1) * 128] * DN
        kh = qkv[:, 1024 + h * 128:1024 + (h + 1) * 128] * DN
        ddq = jnp.dot(qh, pjt, preferred_element_type=_F32)
        ddk = jnp.dot(kh, pjt, preferred_element_type=_F32)
        diagq = 0.5 * jnp.sum(qh * qh, axis=1, keepdims=True)
        diagk = 0.5 * jnp.sum(kh * kh, axis=1, keepdims=True)
        ddq_m = jnp.where(m30, ddq, neg_inf)
        stabq = jnp.max(ddq_m, axis=1, keepdims=True)
        qp = RATIO * (jnp.exp(ddq - diagq - stabq) + 1e-6)
        qp_ref[h] = jnp.where(m30, qp, 0.0)
        sk_ref[h] = jnp.where(m30, ddk - diagk, 0.0)
        mh = jnp.max(jnp.where(m30, ddk, neg_inf))
        kmax = mh if kmax is None else jnp.maximum(kmax, mh)
    kmax_ref[...] = jnp.broadcast_to(kmax, (1, 8, 128))


def _tc_k1(x, w3, b3, pjt, deg, norm):
    first = norm is None
    outs = (
        jax.ShapeDtypeStruct((H, N, 128), _F32),
        jax.ShapeDtypeStruct((H, N, 128), _F32),
        jax.ShapeDtypeStruct((NB, 8, 128), _F32),
        jax.ShapeDtypeStruct((N, 1024), _F32),
        jax.ShapeDtypeStruct((2, N, 512), _F32),
        jax.ShapeDtypeStruct((N, 128), _F32),
    )
    in_specs = [
        pl.BlockSpec((BN, 128), lambda i: (i, 0)),
        pl.BlockSpec((128, 3072), lambda i: (0, 0)),
        pl.BlockSpec((1, 3072), lambda i: (0, 0)),
        pl.BlockSpec((128, 128), lambda i: (0, 0)),
        pl.BlockSpec((BN, 128), lambda i: (i, 0)),
    ]
    args = [x, w3, b3, pjt, deg]
    if not first:
        stats, nw, nb_, nms = norm
        in_specs += [
            pl.BlockSpec((8, 128), lambda i: (0, 0)),
            pl.BlockSpec((1, 128), lambda i: (0, 0)),
            pl.BlockSpec((1, 128), lambda i: (0, 0)),
            pl.BlockSpec((1, 128), lambda i: (0, 0)),
        ]
        args += [stats, nw, nb_, nms]
    out_specs = (
        pl.BlockSpec((H, BN, 128), lambda i: (0, i, 0)),
        pl.BlockSpec((H, BN, 128), lambda i: (0, i, 0)),
        pl.BlockSpec((1, 8, 128), lambda i: (i, 0, 0)),
        pl.BlockSpec((BN, 1024), lambda i: (i, 0)),
        pl.BlockSpec((2, BN, 512), lambda i: (0, i, 0)),
        pl.BlockSpec((BN, 128), lambda i: (i, 0)),
    )
    return pl.pallas_call(
        functools.partial(_k1_body, first),
        grid=(NB,),
        in_specs=in_specs,
        out_specs=out_specs,
        out_shape=outs,
    )(*args)


def _k2_body(refs):
    sk_ref, kmax_ref, v_ref, kvs_ref, ks_ref = refs
    i = pl.program_id(0)
    gmax = jnp.max(kmax_ref[...])
    m30 = _lane_lt(M)

    @pl.when(i == 0)
    def _():
        kvs_ref[...] = jnp.zeros((H, 128, 128), _F32)
        ks_ref[...] = jnp.zeros((H, 128), _F32)

    ks_rows = []
    for h in range(H):
        kp = RATIO * (jnp.exp(sk_ref[h] - gmax) + 1e-6)
        kp = jnp.where(m30, kp, 0.0)
        vh = v_ref[:, h * 128:(h + 1) * 128]
        kvs_ref[h] += lax.dot_general(
            kp, vh, (((0,), (0,)), ((), ())), preferred_element_type=_F32)
        ks_rows.append(jnp.sum(kp, axis=0, keepdims=True))
    ks_ref[...] += jnp.concatenate(ks_rows, axis=0)


def _tc_k2(sk, kmax, v):
    return pl.pallas_call(
        lambda *refs: _k2_body(refs),
        grid=(NB,),
        in_specs=[
            pl.BlockSpec((H, BN, 128), lambda i: (0, i, 0)),
            pl.BlockSpec((NB, 8, 128), lambda i: (0, 0, 0)),
            pl.BlockSpec((BN, 1024), lambda i: (i, 0)),
        ],
        out_specs=(
            pl.BlockSpec((H, 128, 128), lambda i: (0, 0, 0)),
            pl.BlockSpec((H, 128), lambda i: (0, 0)),
        ),
        out_shape=(
            jax.ShapeDtypeStruct((H, 128, 128), _F32),
            jax.ShapeDtypeStruct((H, 128), _F32),
        ),
    )(sk, kmax, v)


def _k3_body(leaky, refs):
    (qp_ref, kvs_ref, ks_ref, agg0_ref, agg1_ref, ab_ref, rb_ref, wo_ref,
     bo_ref, y_ref, stats_ref) = refs
    i = pl.program_id(0)
    rb = rb_ref[...]
    a_col = ab_ref[:, 0:1]
    acc = jnp.zeros((BN, 128), _F32)
    for h in range(H):
        qp = qp_ref[h]
        numer = jnp.dot(qp, kvs_ref[h], preferred_element_type=_F32)
        denom = jnp.sum(qp * ks_ref[h:h + 1, :], axis=1, keepdims=True)
        bh = 1.0 / (1.0 + jnp.exp(-rb[0, h]))
        half_ref = agg0_ref if h < 4 else agg1_ref
        agg_h = half_ref[0][:, (h % 4) * 128:(h % 4 + 1) * 128]
        z = numer / denom + (a_col * bh) * agg_h
        acc += jnp.dot(z, wo_ref[h * 128:(h + 1) * 128, :],
                       preferred_element_type=_F32)
    y = acc + bo_ref[0:1, :]
    if leaky:
        y = jnp.where(y > 0, y, NEG_SLOPE * y)
    y_ref[...] = y
    sy = jnp.sum(y, axis=0, keepdims=True)
    sy2 = jnp.sum(y * y, axis=0, keepdims=True)
    st = jnp.concatenate([sy, sy2, jnp.zeros((6, 128), _F32)], axis=0)

    @pl.when(i == 0)
    def _():
        stats_ref[...] = st

    @pl.when(i > 0)
    def _():
        stats_ref[...] += st


def _tc_k3(qp, kvs, ks, agg, ab, rb, wo, bo, leaky):
    return pl.pallas_call(
        functools.partial(lambda lk, *refs: _k3_body(lk, refs), leaky),
        grid=(NB,),
        in_specs=[
            pl.BlockSpec((H, BN, 128), lambda i: (0, i, 0)),
            pl.BlockSpec((H, 128, 128), lambda i: (0, 0, 0)),
            pl.BlockSpec((H, 128), lambda i: (0, 0)),
            pl.BlockSpec((1, BN, 512), lambda i: (0, i, 0)),
            pl.BlockSpec((1, BN, 512), lambda i: (1, i, 0)),
            pl.BlockSpec((BN, 128), lambda i: (i, 0)),
            pl.BlockSpec((1, 128), lambda i: (0, 0)),
            pl.BlockSpec((1024, 128), lambda i: (0, 0)),
            pl.BlockSpec((1, 128), lambda i: (0, 0)),
        ],
        out_specs=(
            pl.BlockSpec((BN, 128), lambda i: (i, 0)),
            pl.BlockSpec((8, 128), lambda i: (0, 0)),
        ),
        out_shape=(
            jax.ShapeDtypeStruct((N, 128), _F32),
            jax.ShapeDtypeStruct((8, 128), _F32),
        ),
    )(qp, kvs, ks, agg, agg, ab, rb, wo, bo)


def _k4_body(y_ref, stats_ref, nw_ref, nb_ref, nms_ref, out_ref):
    out_ref[...] = _norm_apply(y_ref[...], stats_ref, nw_ref, nb_ref, nms_ref)


def _tc_k4(y, stats, nw, nb_, nms):
    return pl.pallas_call(
        _k4_body,
        grid=(NB,),
        in_specs=[
            pl.BlockSpec((BN, 128), lambda i: (i, 0)),
            pl.BlockSpec((8, 128), lambda i: (0, 0)),
            pl.BlockSpec((1, 128), lambda i: (0, 0)),
            pl.BlockSpec((1, 128), lambda i: (0, 0)),
            pl.BlockSpec((1, 128), lambda i: (0, 0)),
        ],
        out_specs=pl.BlockSpec((BN, 128), lambda i: (i, 0)),
        out_shape=jax.ShapeDtypeStruct((N, 128), _F32),
    )(y, stats, nw, nb_, nms)


def kernel(patch_embs, edge_index, edge_attr, params):
    del edge_attr
    rows = edge_index[0].astype(_I32)
    cols = edge_index[1].astype(_I32)
    pad = jnp.full((EP - E,), SENT, _I32)
    cols_p = jnp.concatenate([cols, pad])
    rows_p = jnp.concatenate([rows, pad])

    deg = _sc_degrees(cols_p, rows_p)
    degT = jnp.pad(
        jnp.stack([deg[:N], deg[2 * NP:2 * NP + N],
                   deg[NP:NP + N], deg[3 * NP:3 * NP + N]], axis=1),
        ((0, 0), (0, 124)))

    norm_keys = ("norm1", "norm2", "norm3")
    conv_keys = ("conv1", "conv2", "conv3")
    x = patch_embs
    norm = None
    stats = None
    for li in range(3):
        p = params[conv_keys[li]]
        w3 = jnp.concatenate([p["Wq"], p["Wk"], p["Wv"]], axis=1)
        b3 = jnp.concatenate([p["bq"], p["bk"], p["bv"]])[None, :]
        pjt = jnp.pad(p["proj"].T, ((0, 0), (0, 128 - M)))
        rb = jnp.pad(p["rb"], (0, 120))[None, :]
        qp, sk, kmax, v, vp, ab = _tc_k1(x, w3, b3, pjt, degT, norm)
        agg = _sc_spmm(vp.reshape(2 * N, 512), cols_p, rows_p)
        agg = agg.reshape(2, AGG_ROWS, 512)
        kvs, ks = _tc_k2(sk, kmax, v)
        y, stats = _tc_k3(qp, kvs, ks, agg, ab, rb, p["Wo"],
                          p["bo"][None, :], leaky=(li < 2))
        np_ = params[norm_keys[li]]
        norm = (stats, np_["w"][None, :], np_["b"][None, :],
                np_["ms"][None, :])
        x = y
    return _tc_k4(x, norm[0], norm[1], norm[2], norm[3])
